# baseline (speedup 1.0000x reference)
import functools

import jax
import jax.numpy as jnp
from jax import lax
from jax.experimental import pallas as pl
from jax.experimental.pallas import tpu as pltpu
from jax.experimental.pallas import tpu_sc as plsc

NC = 2
NS = 16
NW = NC * NS
LANES = 16


def _sc_gather_partial(word, context, W_in, W_out):
  B = word.shape[0]
  V, D = W_in.shape
  BPW = B // NW
  NCH = BPW // 128
  CCH = D // LANES

  word3 = word.astype(jnp.int32).reshape(NW, NCH, 128)
  ctx3 = context.astype(jnp.int32).reshape(NW, NCH, 128)

  mesh = plsc.VectorSubcoreMesh(core_axis_name="c", subcore_axis_name="s",
                                num_cores=NC, num_subcores=NS)

  @functools.partial(
      pl.kernel,
      out_type=[
          jax.ShapeDtypeStruct((B, D), jnp.float32),
          jax.ShapeDtypeStruct((B, LANES), jnp.float32),
      ],
      mesh=mesh,
      compiler_params=pltpu.CompilerParams(use_tc_tiling_on_sc=False),
      scratch_types=[
          pltpu.VMEM((NCH, 128), jnp.int32),
          pltpu.VMEM((NCH, 128), jnp.int32),
          pltpu.VMEM((BPW, D), jnp.float32),
          pltpu.VMEM((BPW, D), jnp.float32),
          pltpu.VMEM((BPW, LANES), jnp.float32),
          pltpu.SemaphoreType.DMA,
      ],
  )
  def sc_kernel(word_hbm, ctx_hbm, win_hbm, wout_hbm,
                emb_hbm, part_hbm,
                idx_u, idx_v, rows_u, rows_v, part_v, sem):
    wid = lax.axis_index("s") * NC + lax.axis_index("c")
    base = wid * BPW

    pltpu.sync_copy(word_hbm.at[wid], idx_u)
    pltpu.sync_copy(ctx_hbm.at[wid], idx_v)

    copies = []
    for j in range(NCH):
      copies.append(pltpu.async_copy(
          win_hbm.at[idx_u.at[j]], rows_u.at[pl.ds(j * 128, 128)], sem))
      copies.append(pltpu.async_copy(
          wout_hbm.at[idx_v.at[j]], rows_v.at[pl.ds(j * 128, 128)], sem))
    for c in copies:
      c.wait()

    pltpu.sync_copy(rows_u, emb_hbm.at[pl.ds(base, BPW)])

    def body(r, _):
      acc = rows_u[r, pl.ds(0, LANES)] * rows_v[r, pl.ds(0, LANES)]
      for c in range(1, CCH):
        acc += rows_u[r, pl.ds(c * LANES, LANES)] * rows_v[r, pl.ds(c * LANES, LANES)]
      part_v[r, :] = acc
      return 0

    lax.fori_loop(0, BPW, body, 0)
    pltpu.sync_copy(part_v, part_hbm.at[pl.ds(base, BPW)])

  return sc_kernel(word3, ctx3, W_in, W_out)


def _tc_loss(partial):
  B = partial.shape[0]

  def body(part_ref, out_ref):
    score = jnp.sum(part_ref[...], axis=1)
    out_ref[0, 0] = -jnp.mean(jax.nn.log_sigmoid(score))

  out = pl.pallas_call(
      body,
      out_shape=jax.ShapeDtypeStruct((1, 1), jnp.float32),
      out_specs=pl.BlockSpec(memory_space=pltpu.SMEM),
  )(partial)
  return out[0, 0]


def kernel(word, context, W_in, W_out):
  embed_u, partial = _sc_gather_partial(word, context, W_in, W_out)
  loss = _tc_loss(partial)
  return (loss, embed_u)


# speedup vs baseline: 1.5642x; 1.5642x over previous
import functools

import jax
import jax.numpy as jnp
from jax import lax
from jax.experimental import pallas as pl
from jax.experimental.pallas import tpu as pltpu
from jax.experimental.pallas import tpu_sc as plsc

NC = 2
NS = 16
NW = NC * NS
KB = 16


def _sc_gather_one(idx2, table):
  NWl, BPW = idx2.shape
  V, D = table.shape
  B = NWl * BPW

  mesh = plsc.VectorSubcoreMesh(core_axis_name="c", subcore_axis_name="s",
                                num_cores=NC, num_subcores=NS)

  @functools.partial(
      pl.kernel,
      out_type=jax.ShapeDtypeStruct((B, D), jnp.float32),
      mesh=mesh,
      compiler_params=pltpu.CompilerParams(needs_layout_passes=False),
      scratch_types=[
          pltpu.VMEM((BPW,), jnp.int32),
          pltpu.VMEM((BPW, D), jnp.float32),
          pltpu.SemaphoreType.DMA,
      ],
  )
  def sc_kernel(idx_hbm, tab_hbm, emb_hbm, idx_v, rows, sem):
    wid = lax.axis_index("s") * NC + lax.axis_index("c")
    base = wid * BPW

    pltpu.sync_copy(idx_hbm.at[wid], idx_v)

    lane = lax.iota(jnp.int32, KB)

    def batch(c, _):
      off = pl.multiple_of(c * KB, KB)
      vec = idx_v[pl.ds(off, KB)]
      copies = []
      for k in range(KB):
        i = jnp.sum(jnp.where(lane == k, vec, 0))
        copies.append(pltpu.async_copy(
            tab_hbm.at[i], rows.at[c * KB + k], sem))
      for cp in copies:
        cp.wait()
      return 0

    lax.fori_loop(0, BPW // KB, batch, 0)

    pltpu.sync_copy(rows, emb_hbm.at[pl.ds(base, BPW)])

  return sc_kernel(idx2, table)


def _tc_loss(emb_u, emb_v):
  def body(u_ref, v_ref, out_ref):
    score = jnp.sum(u_ref[...] * v_ref[...], axis=1)
    out_ref[0, 0] = -jnp.mean(jax.nn.log_sigmoid(score))

  out = pl.pallas_call(
      body,
      out_shape=jax.ShapeDtypeStruct((1, 1), jnp.float32),
      out_specs=pl.BlockSpec(memory_space=pltpu.SMEM),
  )(emb_u, emb_v)
  return out[0, 0]


def kernel(word, context, W_in, W_out):
  B = word.shape[0]
  word2 = word.astype(jnp.int32).reshape(NW, B // NW)
  ctx2 = context.astype(jnp.int32).reshape(NW, B // NW)
  embed_u = _sc_gather_one(word2, W_in)
  embed_v = _sc_gather_one(ctx2, W_out)
  loss = _tc_loss(embed_u, embed_v)
  return (loss, embed_u)


# speedup vs baseline: 1.5970x; 1.0210x over previous
import functools

import jax
import jax.numpy as jnp
from jax import lax
from jax.experimental import pallas as pl
from jax.experimental.pallas import tpu as pltpu
from jax.experimental.pallas import tpu_sc as plsc

NC = 2
NS = 16
NW = NC * NS
KB = 16
LB = 4096


def _tc_pack(table_t):
  D, V = table_t.shape
  nblk = pl.cdiv(V, LB)

  def body(x_ref, out_ref):
    xt = x_ref[...].T
    out_ref[...] = jnp.concatenate(
        [xt[: LB // 2, :], xt[LB // 2 :, :]], axis=1)

  return pl.pallas_call(
      body,
      grid=(nblk,),
      in_specs=[pl.BlockSpec((D, LB), lambda j: (0, j))],
      out_specs=pl.BlockSpec((LB // 2, 2 * D), lambda j: (j, 0)),
      out_shape=jax.ShapeDtypeStruct((nblk * LB // 2, 2 * D), jnp.float32),
  )(table_t)


def _sc_gather_one(idx2, packed):
  NWl, BPW = idx2.shape
  P, D2 = packed.shape
  D = D2 // 2
  B = NWl * BPW

  mesh = plsc.VectorSubcoreMesh(core_axis_name="c", subcore_axis_name="s",
                                num_cores=NC, num_subcores=NS)

  @functools.partial(
      pl.kernel,
      out_type=jax.ShapeDtypeStruct((B, D), jnp.float32),
      mesh=mesh,
      compiler_params=pltpu.CompilerParams(
          use_tc_tiling_on_sc=False, needs_layout_passes=False),
      scratch_types=[
          pltpu.VMEM((BPW,), jnp.int32),
          pltpu.VMEM((BPW, D), jnp.float32),
          pltpu.SemaphoreType.DMA,
      ],
  )
  def sc_kernel(idx_hbm, tab_hbm, emb_hbm, idx_v, rows, sem):
    wid = lax.axis_index("s") * NC + lax.axis_index("c")
    base = wid * BPW

    pltpu.sync_copy(idx_hbm.at[wid], idx_v)

    lane = lax.iota(jnp.int32, KB)

    def batch(c, _):
      off = pl.multiple_of(c * KB, KB)
      vec = idx_v[pl.ds(off, KB)]
      copies = []
      for k in range(KB):
        i = jnp.sum(jnp.where(lane == k, vec, 0))
        p = (i >> 12) * (LB // 2) + (i & (LB // 2 - 1))
        h = pl.multiple_of(((i >> 11) & 1) * D, D)
        copies.append(pltpu.async_copy(
            tab_hbm.at[p, pl.ds(h, D)], rows.at[c * KB + k], sem))
      for cp in copies:
        cp.wait()
      return 0

    lax.fori_loop(0, BPW // KB, batch, 0)

    pltpu.sync_copy(rows, emb_hbm.at[pl.ds(base, BPW)])

  return sc_kernel(idx2, packed)


def _tc_loss(emb_u, emb_v):
  def body(u_ref, v_ref, out_ref):
    score = jnp.sum(u_ref[...] * v_ref[...], axis=1)
    out_ref[0, 0] = -jnp.mean(jax.nn.log_sigmoid(score))

  out = pl.pallas_call(
      body,
      out_shape=jax.ShapeDtypeStruct((1, 1), jnp.float32),
      out_specs=pl.BlockSpec(memory_space=pltpu.SMEM),
  )(emb_u, emb_v)
  return out[0, 0]


def kernel(word, context, W_in, W_out):
  B = word.shape[0]
  word2 = word.astype(jnp.int32).reshape(NW, B // NW)
  ctx2 = context.astype(jnp.int32).reshape(NW, B // NW)
  packed_u = _tc_pack(W_in.T)
  embed_u = _sc_gather_one(word2, packed_u)
  packed_v = _tc_pack(W_out.T)
  embed_v = _sc_gather_one(ctx2, packed_v)
  loss = _tc_loss(embed_u, embed_v)
  return (loss, embed_u)


# speedup vs baseline: 1.9710x; 1.2342x over previous
import functools

import jax
import jax.numpy as jnp
from jax import lax
from jax.experimental import pallas as pl
from jax.experimental.pallas import tpu as pltpu
from jax.experimental.pallas import tpu_sc as plsc

NC = 2
NS = 16
NW = NC * NS
KB = 16
LOG_LB = 13
LB = 1 << LOG_LB


def _tc_pack(table_t):
  D, V = table_t.shape
  nblk = pl.cdiv(V, LB)

  def body(x_ref, out_ref):
    xt = x_ref[...].T
    out_ref[...] = jnp.concatenate(
        [xt[: LB // 2, :], xt[LB // 2 :, :]], axis=1)

  return pl.pallas_call(
      body,
      grid=(nblk,),
      in_specs=[pl.BlockSpec((D, LB), lambda j: (0, j))],
      out_specs=pl.BlockSpec((LB // 2, 2 * D), lambda j: (j, 0)),
      out_shape=jax.ShapeDtypeStruct((nblk * LB // 2, 2 * D), jnp.float32),
  )(table_t)


def _sc_gather_one(idx2, packed):
  NWl, BPW = idx2.shape
  P, D2 = packed.shape
  D = D2 // 2
  B = NWl * BPW

  mesh = plsc.VectorSubcoreMesh(core_axis_name="c", subcore_axis_name="s",
                                num_cores=NC, num_subcores=NS)

  @functools.partial(
      pl.kernel,
      out_type=jax.ShapeDtypeStruct((B, D), jnp.float32),
      mesh=mesh,
      compiler_params=pltpu.CompilerParams(
          use_tc_tiling_on_sc=False, needs_layout_passes=False),
      scratch_types=[
          pltpu.VMEM((BPW,), jnp.int32),
          pltpu.VMEM((BPW, D), jnp.float32),
          pltpu.SemaphoreType.DMA,
      ],
  )
  def sc_kernel(idx_hbm, tab_hbm, emb_hbm, idx_v, rows, sem):
    wid = lax.axis_index("s") * NC + lax.axis_index("c")
    base = wid * BPW

    pltpu.sync_copy(idx_hbm.at[wid], idx_v)

    lane = lax.iota(jnp.int32, KB)

    def batch(c, _):
      off = pl.multiple_of(c * KB, KB)
      vec = idx_v[pl.ds(off, KB)]
      copies = []
      for k in range(KB):
        i = jnp.sum(jnp.where(lane == k, vec, 0))
        p = (i >> LOG_LB) * (LB // 2) + (i & (LB // 2 - 1))
        h = pl.multiple_of(((i >> (LOG_LB - 1)) & 1) * D, D)
        copies.append(pltpu.async_copy(
            tab_hbm.at[p, pl.ds(h, D)], rows.at[c * KB + k], sem))
      for cp in copies:
        cp.wait()
      return 0

    lax.fori_loop(0, BPW // KB, batch, 0)

    pltpu.sync_copy(rows, emb_hbm.at[pl.ds(base, BPW)])

  return sc_kernel(idx2, packed)


def _tc_loss(emb_u, emb_v):
  def body(u_ref, v_ref, out_ref):
    score = jnp.sum(u_ref[...] * v_ref[...], axis=1)
    out_ref[0, 0] = -jnp.mean(jax.nn.log_sigmoid(score))

  out = pl.pallas_call(
      body,
      out_shape=jax.ShapeDtypeStruct((1, 1), jnp.float32),
      out_specs=pl.BlockSpec(memory_space=pltpu.SMEM),
  )(emb_u, emb_v)
  return out[0, 0]


def kernel(word, context, W_in, W_out):
  B = word.shape[0]
  word2 = word.astype(jnp.int32).reshape(NW, B // NW)
  ctx2 = context.astype(jnp.int32).reshape(NW, B // NW)
  packed_u = _tc_pack(W_in.T)
  embed_u = _sc_gather_one(word2, packed_u)
  packed_v = _tc_pack(W_out.T)
  embed_v = _sc_gather_one(ctx2, packed_v)
  loss = _tc_loss(embed_u, embed_v)
  return (loss, embed_u)


# speedup vs baseline: 2.2200x; 1.1263x over previous
import functools

import jax
import jax.numpy as jnp
from jax import lax
from jax.experimental import pallas as pl
from jax.experimental.pallas import tpu as pltpu
from jax.experimental.pallas import tpu_sc as plsc

NC = 2
NS = 16
NW = NC * NS
KB = 16
LOG_LB = 14
LB = 1 << LOG_LB


def _tc_pack(table_t):
  D, V = table_t.shape
  nblk = pl.cdiv(V, LB)

  def body(x_ref, out_ref):
    xt = x_ref[...].T
    out_ref[...] = jnp.concatenate(
        [xt[: LB // 2, :], xt[LB // 2 :, :]], axis=1)

  return pl.pallas_call(
      body,
      grid=(nblk,),
      in_specs=[pl.BlockSpec((D, LB), lambda j: (0, j))],
      out_specs=pl.BlockSpec((LB // 2, 2 * D), lambda j: (j, 0)),
      out_shape=jax.ShapeDtypeStruct((nblk * LB // 2, 2 * D), jnp.float32),
  )(table_t)


def _sc_gather_one(idx2, packed):
  NWl, BPW = idx2.shape
  P, D2 = packed.shape
  D = D2 // 2
  B = NWl * BPW

  mesh = plsc.VectorSubcoreMesh(core_axis_name="c", subcore_axis_name="s",
                                num_cores=NC, num_subcores=NS)

  @functools.partial(
      pl.kernel,
      out_type=jax.ShapeDtypeStruct((B, D), jnp.float32),
      mesh=mesh,
      compiler_params=pltpu.CompilerParams(
          use_tc_tiling_on_sc=False, needs_layout_passes=False),
      scratch_types=[
          pltpu.VMEM((BPW,), jnp.int32),
          pltpu.VMEM((BPW, D), jnp.float32),
          pltpu.SemaphoreType.DMA,
      ],
  )
  def sc_kernel(idx_hbm, tab_hbm, emb_hbm, idx_v, rows, sem):
    wid = lax.axis_index("s") * NC + lax.axis_index("c")
    base = wid * BPW

    pltpu.sync_copy(idx_hbm.at[wid], idx_v)

    lane = lax.iota(jnp.int32, KB)

    def batch(c, _):
      off = pl.multiple_of(c * KB, KB)
      vec = idx_v[pl.ds(off, KB)]
      copies = []
      for k in range(KB):
        i = jnp.sum(jnp.where(lane == k, vec, 0))
        p = (i >> LOG_LB) * (LB // 2) + (i & (LB // 2 - 1))
        h = pl.multiple_of(((i >> (LOG_LB - 1)) & 1) * D, D)
        copies.append(pltpu.async_copy(
            tab_hbm.at[p, pl.ds(h, D)], rows.at[c * KB + k], sem))
      for cp in copies:
        cp.wait()
      return 0

    lax.fori_loop(0, BPW // KB, batch, 0)

    pltpu.sync_copy(rows, emb_hbm.at[pl.ds(base, BPW)])

  return sc_kernel(idx2, packed)


def _tc_loss(emb_u, emb_v):
  def body(u_ref, v_ref, out_ref):
    score = jnp.sum(u_ref[...] * v_ref[...], axis=1)
    out_ref[0, 0] = -jnp.mean(jax.nn.log_sigmoid(score))

  out = pl.pallas_call(
      body,
      out_shape=jax.ShapeDtypeStruct((1, 1), jnp.float32),
      out_specs=pl.BlockSpec(memory_space=pltpu.SMEM),
  )(emb_u, emb_v)
  return out[0, 0]


def kernel(word, context, W_in, W_out):
  B = word.shape[0]
  word2 = word.astype(jnp.int32).reshape(NW, B // NW)
  ctx2 = context.astype(jnp.int32).reshape(NW, B // NW)
  packed_u = _tc_pack(W_in.T)
  embed_u = _sc_gather_one(word2, packed_u)
  packed_v = _tc_pack(W_out.T)
  embed_v = _sc_gather_one(ctx2, packed_v)
  loss = _tc_loss(embed_u, embed_v)
  return (loss, embed_u)


# speedup vs baseline: 2.3533x; 1.0600x over previous
import functools

import jax
import jax.numpy as jnp
from jax import lax
from jax.experimental import pallas as pl
from jax.experimental.pallas import tpu as pltpu
from jax.experimental.pallas import tpu_sc as plsc

NC = 2
NS = 16
NW = NC * NS
KB = 16
LOG_LB = 15
LB = 1 << LOG_LB


def _tc_pack(table_t):
  D, V = table_t.shape
  nblk = pl.cdiv(V, LB)

  def body(x_ref, out_ref):
    xt = x_ref[...].T
    out_ref[...] = jnp.concatenate(
        [xt[: LB // 2, :], xt[LB // 2 :, :]], axis=1)

  return pl.pallas_call(
      body,
      grid=(nblk,),
      in_specs=[pl.BlockSpec((D, LB), lambda j: (0, j))],
      out_specs=pl.BlockSpec((LB // 2, 2 * D), lambda j: (j, 0)),
      out_shape=jax.ShapeDtypeStruct((nblk * LB // 2, 2 * D), jnp.float32),
  )(table_t)


def _sc_gather_one(idx2, packed):
  NWl, BPW = idx2.shape
  P, D2 = packed.shape
  D = D2 // 2
  B = NWl * BPW

  mesh = plsc.VectorSubcoreMesh(core_axis_name="c", subcore_axis_name="s",
                                num_cores=NC, num_subcores=NS)

  @functools.partial(
      pl.kernel,
      out_type=jax.ShapeDtypeStruct((B, D), jnp.float32),
      mesh=mesh,
      compiler_params=pltpu.CompilerParams(
          use_tc_tiling_on_sc=False, needs_layout_passes=False),
      scratch_types=[
          pltpu.VMEM((BPW,), jnp.int32),
          pltpu.VMEM((BPW, D), jnp.float32),
          pltpu.SemaphoreType.DMA,
      ],
  )
  def sc_kernel(idx_hbm, tab_hbm, emb_hbm, idx_v, rows, sem):
    wid = lax.axis_index("s") * NC + lax.axis_index("c")
    base = wid * BPW

    pltpu.sync_copy(idx_hbm.at[wid], idx_v)

    lane = lax.iota(jnp.int32, KB)

    def batch(c, _):
      off = pl.multiple_of(c * KB, KB)
      vec = idx_v[pl.ds(off, KB)]
      copies = []
      for k in range(KB):
        i = jnp.sum(jnp.where(lane == k, vec, 0))
        p = (i >> LOG_LB) * (LB // 2) + (i & (LB // 2 - 1))
        h = pl.multiple_of(((i >> (LOG_LB - 1)) & 1) * D, D)
        copies.append(pltpu.async_copy(
            tab_hbm.at[p, pl.ds(h, D)], rows.at[c * KB + k], sem))
      for cp in copies:
        cp.wait()
      return 0

    lax.fori_loop(0, BPW // KB, batch, 0)

    pltpu.sync_copy(rows, emb_hbm.at[pl.ds(base, BPW)])

  return sc_kernel(idx2, packed)


def _tc_loss(emb_u, emb_v):
  def body(u_ref, v_ref, out_ref):
    score = jnp.sum(u_ref[...] * v_ref[...], axis=1)
    out_ref[0, 0] = -jnp.mean(jax.nn.log_sigmoid(score))

  out = pl.pallas_call(
      body,
      out_shape=jax.ShapeDtypeStruct((1, 1), jnp.float32),
      out_specs=pl.BlockSpec(memory_space=pltpu.SMEM),
  )(emb_u, emb_v)
  return out[0, 0]


def kernel(word, context, W_in, W_out):
  B = word.shape[0]
  word2 = word.astype(jnp.int32).reshape(NW, B // NW)
  ctx2 = context.astype(jnp.int32).reshape(NW, B // NW)
  packed_u = _tc_pack(W_in.T)
  embed_u = _sc_gather_one(word2, packed_u)
  packed_v = _tc_pack(W_out.T)
  embed_v = _sc_gather_one(ctx2, packed_v)
  loss = _tc_loss(embed_u, embed_v)
  return (loss, embed_u)
